# baseline (speedup 1.0000x reference)
import functools

import jax
import jax.numpy as jnp
from jax import lax
from jax.experimental import pallas as pl
from jax.experimental.pallas import tpu as pltpu
from jax.experimental.pallas import tpu_sc as plsc

N = 100000
B = 64
F = 128
L = 16
NC = 2
NS = 16
NW = NC * NS
ROWS_W = N // NW
ROWS_PAD = 3136
CHUNK = 224
NCHUNK = ROWS_PAD // CHUNK
NF = F // L


def _prelude_body(u_ref, wq_ref, bq_ref, wk_ref, bk_ref, c_ref, d_ref):
    qg = jnp.dot(u_ref[...], wq_ref[...], precision=lax.Precision.HIGHEST,
                 preferred_element_type=jnp.float32) + bq_ref[...]
    c_ref[...] = lax.dot_general(qg, wk_ref[...], (((1,), (1,)), ((), ())),
                                 precision=lax.Precision.HIGHEST,
                                 preferred_element_type=jnp.float32)
    d_ref[...] = jnp.dot(qg, bk_ref[...], precision=lax.Precision.HIGHEST,
                         preferred_element_type=jnp.float32)


def _epilogue_body(part_ref, u_ref, wu_ref, bu_ref, out_ref):
    xagg = jnp.sum(part_ref[...], axis=0)
    cat = jnp.concatenate([xagg, u_ref[...]], axis=1)
    out_ref[...] = jnp.dot(cat, wu_ref[...], precision=lax.Precision.HIGHEST,
                           preferred_element_type=jnp.float32) + bu_ref[...]


def _gather16(v, idx):
    return lax.gather(
        v, idx[:, None],
        lax.GatherDimensionNumbers(offset_dims=(), collapsed_slice_dims=(0,),
                                   start_index_map=(0,)),
        slice_sizes=(1,), mode=lax.GatherScatterMode.PROMISE_IN_BOUNDS)


def _sc_agg_body(x_hbm, batch_hbm, c_hbm, d_hbm, out_hbm,
                 c_v, d_v, acc_v, b_v, x_v, sem_c, sem_b, sem_x0, sem_x1):
    cid = lax.axis_index("c")
    sid = lax.axis_index("s")
    wid = sid * NC + cid
    start = wid * ROWS_W
    end = start + ROWS_W
    a0 = jnp.minimum((start // 8) * 8, N - ROWS_PAD)

    cp_c = pltpu.make_async_copy(c_hbm, c_v, sem_c)
    cp_c.start()
    cp_b = pltpu.make_async_copy(batch_hbm.at[pl.ds(a0, ROWS_PAD)],
                                 b_v.at[pl.ds(0, ROWS_PAD)], sem_b)
    cp_b.start()
    cp_d = pltpu.make_async_copy(d_hbm, d_v.at[pl.ds(0, B)], sem_c)

    sems = (sem_x0, sem_x1)

    def start_chunk(c):
        buf = c % 2
        cp = pltpu.make_async_copy(
            x_hbm.at[pl.ds(a0 + c * CHUNK, CHUNK), :], x_v.at[buf],
            sems[buf])
        cp.start()
        return cp

    start_chunk(0)
    start_chunk(1)

    zv = jnp.zeros((L,), jnp.float32)
    lane = lax.iota(jnp.int32, L)
    bfly = [lane ^ k for k in (1, 2, 4, 8)]
    zero_idx = lane ^ lane
    mk1 = (lane & 1) != 0
    mk2 = (lane & 2) != 0
    quad_idx = [zero_idx + i for i in range(4)]

    def merge(u, v, s, m):
        return jnp.where(m, v + _gather16(v, lane ^ s),
                         u + _gather16(u, lane ^ s))

    def zero_body(i, _):
        for f in range(NF):
            acc_v[i, pl.ds(L * f, L)] = zv
        return 0

    lax.fori_loop(0, B, zero_body, 0)

    cp_c.wait()
    cp_d.start()
    cp_b.wait()
    cp_d.wait()

    def wait_chunk(buf):
        pltpu.make_async_copy(
            x_hbm.at[pl.ds(a0, CHUNK), :], x_v.at[buf], sems[buf]).wait()

    def row_body(r, _, buf, c):
        g = a0 + c * CHUNK + r
        b = b_v[pl.ds(c * CHUNK + r, L)][0]
        xr = []
        prods = []
        for f in range(NF):
            xv = x_v[buf, r, pl.ds(L * f, L)]
            xr.append(xv)
            prods.append(xv * c_v[b, pl.ds(L * f, L)])
        while len(prods) > 1:
            prods = [prods[i] + prods[i + 1] for i in range(0, len(prods), 2)]
        p = prods[0]
        for idx in bfly:
            p = p + _gather16(p, idx)
        dbc = _gather16(d_v[pl.ds(b, L)], zero_idx)
        sv = p + dbc
        own = jnp.where((g >= start) & (g < end), 1.0, 0.0)
        av = jnp.full((L,), own, jnp.float32) / (1.0 + jnp.exp(-sv))
        for f in range(NF):
            plsc.addupdate(acc_v.at[b, pl.ds(L * f, L)], av * xr[f])
        return 0

    def group_body(grp, _, buf, c):
        base = c * CHUNK + grp * L
        gstart = a0 + base
        bv16 = b_v[pl.ds(base, L)]
        b0 = bv16[0]
        uniform = ((b0 == bv16[15]) & (gstart >= start)
                   & (gstart + L <= end))

        @pl.when(uniform)
        def _fast():
            cr = [c_v[b0, pl.ds(L * f, L)] for f in range(NF)]
            dbc = _gather16(d_v[pl.ds(b0, L)], zero_idx)
            sacc = [zv] * NF
            for q in range(4):
                xs = []
                ps = []
                for i in range(4):
                    r = grp * L + q * 4 + i
                    xv = [x_v[buf, r, pl.ds(L * f, L)] for f in range(NF)]
                    prods = [xv[f] * cr[f] for f in range(NF)]
                    while len(prods) > 1:
                        prods = [prods[j] + prods[j + 1]
                                 for j in range(0, len(prods), 2)]
                    xs.append(xv)
                    ps.append(prods[0])
                w = merge(merge(ps[0], ps[1], 1, mk1),
                          merge(ps[2], ps[3], 1, mk1), 2, mk2)
                w = w + _gather16(w, bfly[2])
                w = w + _gather16(w, bfly[3])
                aw = 1.0 / (1.0 + jnp.exp(-(w + dbc)))
                for i in range(4):
                    av = _gather16(aw, quad_idx[i])
                    for f in range(NF):
                        sacc[f] = sacc[f] + av * xs[i][f]
            for f in range(NF):
                plsc.addupdate(acc_v.at[b0, pl.ds(L * f, L)], sacc[f])

        @pl.when(jnp.logical_not(uniform))
        def _slow():
            lax.fori_loop(
                0, L,
                lambda i, s: row_body(grp * L + i, s, buf=buf, c=c), 0)

        return 0

    def pair_body(cpair, _):
        c0 = cpair * 2
        for half in (0, 1):
            c = c0 + half
            wait_chunk(half)

            @plsc.parallel_loop(0, CHUNK // L, step=1, unroll=2)
            def _groups(grp, buf=half, c=c):
                group_body(grp, 0, buf=buf, c=c)

            @pl.when(c + 2 < NCHUNK)
            def _():
                buf = half
                cp = pltpu.make_async_copy(
                    x_hbm.at[pl.ds(a0 + (c + 2) * CHUNK, CHUNK), :],
                    x_v.at[buf], sems[buf])
                cp.start()
        return 0

    lax.fori_loop(0, NCHUNK // 2, pair_body, 0)

    pltpu.sync_copy(acc_v, out_hbm.at[wid])


_sc_agg = functools.partial(
    pl.kernel,
    out_type=jax.ShapeDtypeStruct((NW, B, F), jnp.float32),
    mesh=plsc.VectorSubcoreMesh(core_axis_name="c", subcore_axis_name="s",
                                num_cores=NC, num_subcores=NS),
    scratch_types=[
        pltpu.VMEM((B, F), jnp.float32),
        pltpu.VMEM((B + L,), jnp.float32),
        pltpu.VMEM((B, F), jnp.float32),
        pltpu.VMEM((ROWS_PAD + L,), jnp.int32),
        pltpu.VMEM((2, CHUNK, F), jnp.float32),
        pltpu.SemaphoreType.DMA,
        pltpu.SemaphoreType.DMA,
        pltpu.SemaphoreType.DMA,
        pltpu.SemaphoreType.DMA,
    ],
)(_sc_agg_body)


def kernel(x, u, batch, Wk, bk, Wq, bq, Wu, bu):
    batch32 = batch.astype(jnp.int32)
    C, d = pl.pallas_call(
        _prelude_body,
        out_shape=[
            jax.ShapeDtypeStruct((B, F), jnp.float32),
            jax.ShapeDtypeStruct((B, 1), jnp.float32),
        ],
    )(u, Wq, bq.reshape(1, F), Wk, bk.reshape(F, 1))
    part = _sc_agg(x, batch32, C, d.reshape(B))
    out = pl.pallas_call(
        _epilogue_body,
        out_shape=jax.ShapeDtypeStruct((B, F), jnp.float32),
    )(part, u, Wu, bu.reshape(1, F))
    return out


# speedup vs baseline: 1.0629x; 1.0629x over previous
import functools

import jax
import jax.numpy as jnp
from jax import lax
from jax.experimental import pallas as pl
from jax.experimental.pallas import tpu as pltpu
from jax.experimental.pallas import tpu_sc as plsc

N = 100000
B = 64
F = 128
L = 16
NC = 2
NS = 16
NW = NC * NS
ROWS_W = N // NW
ROWS_PAD = 3136
CHUNK = 224
NCHUNK = ROWS_PAD // CHUNK
NF = F // L


def _prelude_body(u_ref, wq_ref, bq_ref, wk_ref, bk_ref, c_ref, d_ref):
    qg = jnp.dot(u_ref[...], wq_ref[...], precision=lax.Precision.HIGHEST,
                 preferred_element_type=jnp.float32) + bq_ref[...]
    c_ref[...] = lax.dot_general(qg, wk_ref[...], (((1,), (1,)), ((), ())),
                                 precision=lax.Precision.HIGHEST,
                                 preferred_element_type=jnp.float32)
    d_ref[...] = jnp.dot(qg, bk_ref[...], precision=lax.Precision.HIGHEST,
                         preferred_element_type=jnp.float32)


def _epilogue_body(part_ref, u_ref, wu_ref, bu_ref, out_ref):
    xagg = jnp.sum(part_ref[...], axis=0)
    cat = jnp.concatenate([xagg, u_ref[...]], axis=1)
    out_ref[...] = jnp.dot(cat, wu_ref[...], precision=lax.Precision.HIGHEST,
                           preferred_element_type=jnp.float32) + bu_ref[...]


def _gather16(v, idx):
    return lax.gather(
        v, idx[:, None],
        lax.GatherDimensionNumbers(offset_dims=(), collapsed_slice_dims=(0,),
                                   start_index_map=(0,)),
        slice_sizes=(1,), mode=lax.GatherScatterMode.PROMISE_IN_BOUNDS)


def _sc_agg_body(x_hbm, batch_hbm, c_hbm, d_hbm, out_hbm,
                 c_v, d_v, acc_v, b_v, x_v, sem_c, sem_b, sem_x0, sem_x1):
    cid = lax.axis_index("c")
    sid = lax.axis_index("s")
    wid = sid * NC + cid
    start = wid * ROWS_W
    end = start + ROWS_W
    a0 = jnp.minimum((start // 8) * 8, N - ROWS_PAD)

    cp_c = pltpu.make_async_copy(c_hbm, c_v, sem_c)
    cp_c.start()
    cp_b = pltpu.make_async_copy(batch_hbm.at[pl.ds(a0, ROWS_PAD)],
                                 b_v.at[pl.ds(0, ROWS_PAD)], sem_b)
    cp_b.start()
    cp_d = pltpu.make_async_copy(d_hbm, d_v.at[pl.ds(0, B)], sem_c)

    sems = (sem_x0, sem_x1)

    def start_chunk(c):
        buf = c % 2
        cp = pltpu.make_async_copy(
            x_hbm.at[pl.ds(a0 + c * CHUNK, CHUNK), :], x_v.at[buf],
            sems[buf])
        cp.start()
        return cp

    start_chunk(0)
    start_chunk(1)

    zv = jnp.zeros((L,), jnp.float32)
    lane = lax.iota(jnp.int32, L)
    bfly = [lane ^ k for k in (1, 2, 4, 8)]
    zero_idx = lane ^ lane
    mk1 = (lane & 1) != 0
    mk2 = (lane & 2) != 0
    quad_idx = [zero_idx + i for i in range(4)]

    def merge(u, v, s, m):
        return jnp.where(m, v + _gather16(v, lane ^ s),
                         u + _gather16(u, lane ^ s))

    def zero_body(i, _):
        for f in range(NF):
            acc_v[i, pl.ds(L * f, L)] = zv
        return 0

    lax.fori_loop(0, B, zero_body, 0)

    cp_c.wait()
    cp_d.start()
    cp_b.wait()
    cp_d.wait()

    def wait_chunk(buf):
        pltpu.make_async_copy(
            x_hbm.at[pl.ds(a0, CHUNK), :], x_v.at[buf], sems[buf]).wait()

    def row_body(r, _, buf, c):
        g = a0 + c * CHUNK + r
        b = b_v[pl.ds(c * CHUNK + r, L)][0]
        xr = []
        prods = []
        for f in range(NF):
            xv = x_v[buf, r, pl.ds(L * f, L)]
            xr.append(xv)
            prods.append(xv * c_v[b, pl.ds(L * f, L)])
        while len(prods) > 1:
            prods = [prods[i] + prods[i + 1] for i in range(0, len(prods), 2)]
        p = prods[0]
        for idx in bfly:
            p = p + _gather16(p, idx)
        dbc = _gather16(d_v[pl.ds(b, L)], zero_idx)
        sv = p + dbc
        own = jnp.where((g >= start) & (g < end), 1.0, 0.0)
        av = jnp.full((L,), own, jnp.float32) / (1.0 + jnp.exp(-sv))
        for f in range(NF):
            plsc.addupdate(acc_v.at[b, pl.ds(L * f, L)], av * xr[f])
        return 0

    def group_body(grp, _, buf, c):
        base = c * CHUNK + grp * L
        gstart = a0 + base
        bv16 = b_v[pl.ds(base, L)]
        b0 = bv16[0]
        uniform = ((b0 == bv16[15]) & (gstart >= start)
                   & (gstart + L <= end))

        @pl.when(uniform)
        def _fast():
            cr = [c_v[b0, pl.ds(L * f, L)] for f in range(NF)]
            dbc = _gather16(d_v[pl.ds(b0, L)], zero_idx)
            sacc = [zv] * NF
            for q in range(4):
                xs = []
                ps = []
                for i in range(4):
                    r = grp * L + q * 4 + i
                    xv = [x_v[buf, r, pl.ds(L * f, L)] for f in range(NF)]
                    prods = [xv[f] * cr[f] for f in range(NF)]
                    while len(prods) > 1:
                        prods = [prods[j] + prods[j + 1]
                                 for j in range(0, len(prods), 2)]
                    xs.append(xv)
                    ps.append(prods[0])
                w = merge(merge(ps[0], ps[1], 1, mk1),
                          merge(ps[2], ps[3], 1, mk1), 2, mk2)
                w = w + _gather16(w, bfly[2])
                w = w + _gather16(w, bfly[3])
                aw = 1.0 / (1.0 + jnp.exp(-(w + dbc)))
                for i in range(4):
                    av = _gather16(aw, quad_idx[i])
                    for f in range(NF):
                        sacc[f] = sacc[f] + av * xs[i][f]
            for f in range(NF):
                plsc.addupdate(acc_v.at[b0, pl.ds(L * f, L)], sacc[f])

        @pl.when(jnp.logical_not(uniform))
        def _slow():
            lax.fori_loop(
                0, L,
                lambda i, s: row_body(grp * L + i, s, buf=buf, c=c), 0)

        return 0

    def pair_body(cpair, _):
        c0 = cpair * 2
        for half in (0, 1):
            c = c0 + half
            wait_chunk(half)

            @plsc.parallel_loop(0, CHUNK // L, step=1)
            def _groups(grp, buf=half, c=c):
                group_body(grp, 0, buf=buf, c=c)

            @pl.when(c + 2 < NCHUNK)
            def _():
                buf = half
                cp = pltpu.make_async_copy(
                    x_hbm.at[pl.ds(a0 + (c + 2) * CHUNK, CHUNK), :],
                    x_v.at[buf], sems[buf])
                cp.start()
        return 0

    lax.fori_loop(0, NCHUNK // 2, pair_body, 0)

    pltpu.sync_copy(acc_v, out_hbm.at[wid])


_sc_agg = functools.partial(
    pl.kernel,
    out_type=jax.ShapeDtypeStruct((NW, B, F), jnp.float32),
    mesh=plsc.VectorSubcoreMesh(core_axis_name="c", subcore_axis_name="s",
                                num_cores=NC, num_subcores=NS),
    scratch_types=[
        pltpu.VMEM((B, F), jnp.float32),
        pltpu.VMEM((B + L,), jnp.float32),
        pltpu.VMEM((B, F), jnp.float32),
        pltpu.VMEM((ROWS_PAD + L,), jnp.int32),
        pltpu.VMEM((2, CHUNK, F), jnp.float32),
        pltpu.SemaphoreType.DMA,
        pltpu.SemaphoreType.DMA,
        pltpu.SemaphoreType.DMA,
        pltpu.SemaphoreType.DMA,
    ],
)(_sc_agg_body)


def kernel(x, u, batch, Wk, bk, Wq, bq, Wu, bu):
    batch32 = batch.astype(jnp.int32)
    C, d = pl.pallas_call(
        _prelude_body,
        out_shape=[
            jax.ShapeDtypeStruct((B, F), jnp.float32),
            jax.ShapeDtypeStruct((B, 1), jnp.float32),
        ],
    )(u, Wq, bq.reshape(1, F), Wk, bk.reshape(F, 1))
    part = _sc_agg(x, batch32, C, d.reshape(B))
    out = pl.pallas_call(
        _epilogue_body,
        out_shape=jax.ShapeDtypeStruct((B, F), jnp.float32),
    )(part, u, Wu, bu.reshape(1, F))
    return out
